# baseline (speedup 1.0000x reference)
import dataclasses
import functools

import jax
import jax.numpy as jnp
from jax import lax
from jax.experimental import pallas as pl
from jax.experimental.pallas import tpu as pltpu
from jax.experimental.pallas import tpu_sc as plsc

N = 10000
E = 320000
D = 128
R = 7
T = 4
KEYS = R * N
KEYS_PAD = 70144
C = 128
CHUNKS = E // C
NC = 2
NS = 16
NW = NC * NS
ITERS = (CHUNKS + NW - 1) // NW
PAIRS = (ITERS + 1) // 2
ITERS_CNT = (CHUNKS + NS - 1) // NS
PAIRS_CNT = (ITERS_CNT + 1) // 2
ELEMS_PER_TILE = KEYS_PAD // NS
NPT = 624

_mesh = plsc.VectorSubcoreMesh(core_axis_name="c", subcore_axis_name="s")
_f32 = jnp.float32
_i32 = jnp.int32

_sc_params = pltpu.CompilerParams()
if "needs_layout_passes" in pltpu.CompilerParams.__dataclass_fields__:
    _sc_params = dataclasses.replace(_sc_params, needs_layout_passes=False)


def _tc_pre_body(xs_ref, xt_ref, wr_ref, wroot_ref, broot_ref, tnt_ref,
                 src_ref, dst_ref, typ_ref,
                 z_ref, root_ref, gk_ref, ck_ref):
    xs = xs_ref[...]
    dn = (((1,), (1,)), ((), ()))
    for r in range(R):
        z_ref[r] = lax.dot_general(xs, wr_ref[r], dn,
                                   precision=lax.Precision.HIGHEST,
                                   preferred_element_type=_f32)
    xt = xt_ref[...]
    tt = tnt_ref[...]
    acc = jnp.zeros_like(xt)
    for i in range(T):
        v = lax.dot_general(xt, wroot_ref[i], dn,
                            precision=lax.Precision.HIGHEST,
                            preferred_element_type=_f32) + broot_ref[i][None, :]
        acc = acc + jnp.where(tt == i, 1.0, 0.0).astype(_f32) * v
    root_ref[...] = acc

    @pl.when(pl.program_id(0) == 0)
    def _():
        t = typ_ref[...]
        gk_ref[...] = t * N + src_ref[...]
        ck_ref[...] = t * N + dst_ref[...]


def _tc_precompute(x_src, x_target, W_rel, W_root, b_root, tnt2d,
                   src2d, dst2d, typ2d):
    nb = 10
    bn = N // nb
    return pl.pallas_call(
        _tc_pre_body,
        grid=(nb,),
        in_specs=[
            pl.BlockSpec((bn, D), lambda i: (i, 0)),
            pl.BlockSpec((bn, D), lambda i: (i, 0)),
            pl.BlockSpec((R, D, D), lambda i: (0, 0, 0)),
            pl.BlockSpec((T, D, D), lambda i: (0, 0, 0)),
            pl.BlockSpec((T, D), lambda i: (0, 0)),
            pl.BlockSpec((bn, 1), lambda i: (i, 0)),
            pl.BlockSpec((CHUNKS, C), lambda i: (0, 0)),
            pl.BlockSpec((CHUNKS, C), lambda i: (0, 0)),
            pl.BlockSpec((CHUNKS, C), lambda i: (0, 0)),
        ],
        out_specs=[
            pl.BlockSpec((R, bn, D), lambda i: (0, i, 0)),
            pl.BlockSpec((bn, D), lambda i: (i, 0)),
            pl.BlockSpec((CHUNKS, C), lambda i: (0, 0)),
            pl.BlockSpec((CHUNKS, C), lambda i: (0, 0)),
        ],
        out_shape=[
            jax.ShapeDtypeStruct((R, N, D), _f32),
            jax.ShapeDtypeStruct((N, D), _f32),
            jax.ShapeDtypeStruct((CHUNKS, C), _i32),
            jax.ShapeDtypeStruct((CHUNKS, C), _i32),
        ],
    )(x_src, x_target, W_rel, W_root, b_root, tnt2d, src2d, dst2d, typ2d)


def _sc_body(gkey_hbm, ckey_hbm, dst_hbm, z_hbm, out_hbm,
             gk0, gk1, ck0, ck1, dv0, dv1, sd0, sd1, sc0, sc1, r0, r1,
             onesv, zerov,
             si0, si1, ss0, ss1, sz0, sz1, so0, so1,
             cnt_sh, acc_sh):
    GK = (gk0, gk1)
    CK = (ck0, ck1)
    DV = (dv0, dv1)
    SD = (sd0, sd1)
    SCV = (sc0, sc1)
    RW = (r0, r1)
    SI = (si0, si1)
    SS = (ss0, ss1)
    SZ = (sz0, sz1)
    SO = (so0, so1)
    core = lax.axis_index("c")
    sub = lax.axis_index("s")
    wid = sub * NC + core
    ones16 = jnp.full((16,), 1.0, _f32)
    zero_row = jnp.zeros((16,), _f32)

    @pl.loop(0, C, step=16)
    def _(g):
        onesv[pl.ds(g, 16)] = ones16

    @pl.loop(0, ELEMS_PER_TILE, step=16)
    def _(g):
        zerov[pl.ds(g, 16)] = zero_row

    @pl.loop(0, C)
    def _(j):
        for k in range(8):
            r0[j, pl.ds(k * 16, 16)] = zero_row

    pltpu.sync_copy(zerov, cnt_sh.at[pl.ds(sub * ELEMS_PER_TILE, ELEMS_PER_TILE)])
    for k in range(4):
        pltpu.sync_copy(r0, acc_sh.at[pl.ds(sub * NPT + k * 128, 128)])
    pltpu.sync_copy(r0.at[pl.ds(0, 112)],
                    acc_sh.at[pl.ds(sub * NPT + 512, 112)])

    @pl.when(sub == NS - 1)
    def _():
        pltpu.sync_copy(r0.at[pl.ds(0, 16)], acc_sh.at[pl.ds(NS * NPT, 16)])

    plsc.subcore_barrier()

    for i0 in (0, 1):
        pltpu.async_copy(ckey_hbm.at[pl.ds((sub + i0 * NS) * C, C)],
                         CK[i0], SI[i0])

    @pl.loop(0, PAIRS_CNT)
    def _(p):
        for off in (0, 1):
            b = off
            i = 2 * p + off
            cid = sub + i * NS
            cid2 = cid + 2 * NS

            @pl.when(cid < CHUNKS)
            def _():
                pltpu.make_async_copy(ckey_hbm.at[pl.ds(cid * C, C)],
                                      CK[b], SI[b]).wait()
                pltpu.async_copy(onesv, cnt_sh.at[CK[b]], SO[b], add=True)

            @pl.when(cid2 < CHUNKS)
            def _():
                pltpu.make_async_copy(onesv, cnt_sh.at[CK[b]], SO[b]).wait()
                pltpu.async_copy(ckey_hbm.at[pl.ds(cid2 * C, C)], CK[b], SI[b])

    pltpu.make_async_copy(onesv, cnt_sh.at[CK[0]], SO[0]).wait()
    pltpu.make_async_copy(onesv, cnt_sh.at[CK[1]], SO[1]).wait()

    plsc.subcore_barrier()

    def idx_issue(i, b):
        cid = wid + i * NW

        @pl.when(cid < CHUNKS)
        def _():
            base = cid * C
            pltpu.async_copy(gkey_hbm.at[pl.ds(base, C)], GK[b], SI[b])
            pltpu.async_copy(ckey_hbm.at[pl.ds(base, C)], CK[b], SI[b])
            pltpu.async_copy(dst_hbm.at[pl.ds(base, C)], DV[b], SI[b])

    def gathers_issue(i, b):
        cid = wid + i * NW

        @pl.when(cid < CHUNKS)
        def _():
            base = cid * C
            pltpu.make_async_copy(gkey_hbm.at[pl.ds(base, C)], GK[b], SI[b]).wait()
            pltpu.make_async_copy(ckey_hbm.at[pl.ds(base, C)], CK[b], SI[b]).wait()
            pltpu.make_async_copy(dst_hbm.at[pl.ds(base, C)], DV[b], SI[b]).wait()

            @pl.when(i >= 2)
            def _():
                pltpu.make_async_copy(RW[b], acc_sh.at[SD[b]], SO[b]).wait()

            pltpu.async_copy(cnt_sh.at[CK[b]], SCV[b], SS[b])
            pltpu.async_copy(z_hbm.at[GK[b]], RW[b], SZ[b])

    def process(i, b):
        cid = wid + i * NW

        @pl.when(cid < CHUNKS)
        def _():
            pltpu.make_async_copy(cnt_sh.at[CK[b]], SCV[b], SS[b]).wait()
            pltpu.make_async_copy(z_hbm.at[GK[b]], RW[b], SZ[b]).wait()

            @pl.loop(0, C, step=16)
            def _(g):
                cval = SCV[b][pl.ds(g, 16)]
                SCV[b][pl.ds(g, 16)] = 1.0 / jnp.maximum(cval, 1.0)

            @pl.loop(0, C, step=16)
            def _(g):
                SD[b][pl.ds(g, 16)] = DV[b][pl.ds(g, 16)]

        idx_issue(i + 2, b)

        @pl.when(cid < CHUNKS)
        def _():
            @pl.loop(0, C)
            def _(j):
                jj = lax.broadcast(j, (16,))
                s16 = plsc.load_gather(SCV[b], [jj])
                for k in range(8):
                    RW[b][j, pl.ds(k * 16, 16)] = RW[b][j, pl.ds(k * 16, 16)] * s16

            pltpu.async_copy(RW[b], acc_sh.at[SD[b]], SO[b], add=True)

    idx_issue(0, 0)
    idx_issue(1, 1)
    gathers_issue(0, 0)

    @pl.loop(0, PAIRS)
    def _(p):
        for off in (0, 1):
            i = 2 * p + off
            gathers_issue(i + 1, (off + 1) % 2)
            process(i, off)

    pltpu.make_async_copy(RW[0], acc_sh.at[SD[0]], SO[0]).wait()
    pltpu.make_async_copy(RW[1], acc_sh.at[SD[1]], SO[1]).wait()

    plsc.subcore_barrier()
    for k in range(4):
        pltpu.sync_copy(acc_sh.at[pl.ds(sub * NPT + k * 128, 128)], r0)
        pltpu.sync_copy(r0, out_hbm.at[core, pl.ds(sub * NPT + k * 128, 128)])
    pltpu.sync_copy(acc_sh.at[pl.ds(sub * NPT + 512, 112)], r0.at[pl.ds(0, 112)])
    pltpu.sync_copy(r0.at[pl.ds(0, 112)],
                    out_hbm.at[core, pl.ds(sub * NPT + 512, 112)])

    @pl.when(sub == NS - 1)
    def _():
        pltpu.sync_copy(acc_sh.at[pl.ds(NS * NPT, 16)], r0.at[pl.ds(0, 16)])
        pltpu.sync_copy(r0.at[pl.ds(0, 16)],
                        out_hbm.at[core, pl.ds(NS * NPT, 16)])


_sc_run = pl.kernel(
    _sc_body,
    out_type=jax.ShapeDtypeStruct((NC, N, D), _f32),
    mesh=_mesh,
    scratch_types=[
        pltpu.VMEM((C,), _i32),
        pltpu.VMEM((C,), _i32),
        pltpu.VMEM((C,), _i32),
        pltpu.VMEM((C,), _i32),
        pltpu.VMEM((C,), _i32),
        pltpu.VMEM((C,), _i32),
        pltpu.VMEM((C,), _i32),
        pltpu.VMEM((C,), _i32),
        pltpu.VMEM((C,), _f32),
        pltpu.VMEM((C,), _f32),
        pltpu.VMEM((C, D), _f32),
        pltpu.VMEM((C, D), _f32),
        pltpu.VMEM((C,), _f32),
        pltpu.VMEM((ELEMS_PER_TILE,), _f32),
        pltpu.SemaphoreType.DMA,
        pltpu.SemaphoreType.DMA,
        pltpu.SemaphoreType.DMA,
        pltpu.SemaphoreType.DMA,
        pltpu.SemaphoreType.DMA,
        pltpu.SemaphoreType.DMA,
        pltpu.SemaphoreType.DMA,
        pltpu.SemaphoreType.DMA,
        pltpu.VMEM_SHARED((KEYS_PAD,), _f32),
        pltpu.VMEM_SHARED((N, D), _f32),
    ],
    compiler_params=_sc_params,
)


def _tc_final_body(part_ref, root_ref, out_ref):
    p = part_ref[...]
    out_ref[...] = p[0] + p[1] + root_ref[...]


def _tc_final(parts, root):
    nb = 10
    bn = N // nb
    return pl.pallas_call(
        _tc_final_body,
        grid=(nb,),
        in_specs=[
            pl.BlockSpec((NC, bn, D), lambda i: (0, i, 0)),
            pl.BlockSpec((bn, D), lambda i: (i, 0)),
        ],
        out_specs=pl.BlockSpec((bn, D), lambda i: (i, 0)),
        out_shape=jax.ShapeDtypeStruct((N, D), _f32),
    )(parts, root)


def kernel(x_src, x_target, edge_index, edge_type, target_node_type,
           src_node_type, W_rel, W_root, b_root):
    src2d = edge_index[0].astype(_i32).reshape(CHUNKS, C)
    dst2d = edge_index[1].astype(_i32).reshape(CHUNKS, C)
    typ2d = edge_type.astype(_i32).reshape(CHUNKS, C)
    tnt2d = target_node_type.astype(_i32).reshape(N, 1)

    z, root, gkey2d, ckey2d = _tc_precompute(
        x_src.astype(_f32), x_target.astype(_f32), W_rel.astype(_f32),
        W_root.astype(_f32), b_root.astype(_f32), tnt2d, src2d, dst2d, typ2d)
    z = z.reshape(KEYS, D)
    gkey = gkey2d.reshape(E)
    ckey = ckey2d.reshape(E)
    dst = dst2d.reshape(E)

    parts = _sc_run(gkey, ckey, dst, z)
    return _tc_final(parts, root)


# speedup vs baseline: 1.1366x; 1.1366x over previous
import dataclasses
import functools

import jax
import jax.numpy as jnp
from jax import lax
from jax.experimental import pallas as pl
from jax.experimental.pallas import tpu as pltpu
from jax.experimental.pallas import tpu_sc as plsc

N = 10000
E = 320000
D = 128
R = 7
T = 4
KEYS = R * N
KEYS_PAD = 70144
C = 128
CHUNKS = E // C
NC = 2
NS = 16
NW = NC * NS
ITERS = (CHUNKS + NW - 1) // NW
PAIRS = (ITERS + 1) // 2
ITERS_CNT = (CHUNKS + NS - 1) // NS
PAIRS_CNT = (ITERS_CNT + 1) // 2
ELEMS_PER_TILE = KEYS_PAD // NS
NPT = 624

_mesh = plsc.VectorSubcoreMesh(core_axis_name="c", subcore_axis_name="s")
_f32 = jnp.float32
_i32 = jnp.int32

_sc_params = pltpu.CompilerParams()
if "needs_layout_passes" in pltpu.CompilerParams.__dataclass_fields__:
    _sc_params = dataclasses.replace(_sc_params, needs_layout_passes=False)


def _tc_pre_body(xs_ref, xt_ref, wr_ref, wroot_ref, broot_ref, tnt_ref,
                 z_ref, root_ref):
    xs = xs_ref[...]
    dn = (((1,), (1,)), ((), ()))
    for r in range(R):
        z_ref[r] = lax.dot_general(xs, wr_ref[r], dn,
                                   precision=lax.Precision.HIGHEST,
                                   preferred_element_type=_f32)
    xt = xt_ref[...]
    tt = tnt_ref[...]
    acc = jnp.zeros_like(xt)
    for i in range(T):
        v = lax.dot_general(xt, wroot_ref[i], dn,
                            precision=lax.Precision.HIGHEST,
                            preferred_element_type=_f32) + broot_ref[i][None, :]
        acc = acc + jnp.where(tt == i, 1.0, 0.0).astype(_f32) * v
    root_ref[...] = acc


def _tc_precompute(x_src, x_target, W_rel, W_root, b_root, tnt2d):
    nb = 10
    bn = N // nb
    return pl.pallas_call(
        _tc_pre_body,
        grid=(nb,),
        in_specs=[
            pl.BlockSpec((bn, D), lambda i: (i, 0)),
            pl.BlockSpec((bn, D), lambda i: (i, 0)),
            pl.BlockSpec((R, D, D), lambda i: (0, 0, 0)),
            pl.BlockSpec((T, D, D), lambda i: (0, 0, 0)),
            pl.BlockSpec((T, D), lambda i: (0, 0)),
            pl.BlockSpec((bn, 1), lambda i: (i, 0)),
        ],
        out_specs=[
            pl.BlockSpec((R, bn, D), lambda i: (0, i, 0)),
            pl.BlockSpec((bn, D), lambda i: (i, 0)),
        ],
        out_shape=[
            jax.ShapeDtypeStruct((R, N, D), _f32),
            jax.ShapeDtypeStruct((N, D), _f32),
        ],
    )(x_src, x_target, W_rel, W_root, b_root, tnt2d)


def _tc_keys_body(src_ref, dst_ref, typ_ref, gk_ref, ck_ref):
    t = typ_ref[...]
    gk_ref[...] = t * N + src_ref[...]
    ck_ref[...] = t * N + dst_ref[...]


def _tc_keys(src2d, dst2d, typ2d):
    return pl.pallas_call(
        _tc_keys_body,
        out_shape=[jax.ShapeDtypeStruct((CHUNKS, C), _i32)] * 2,
    )(src2d, dst2d, typ2d)


def _sc_count_body(ckey_hbm, out_hbm, kv0, kv1, onesv, zerov, cnt_sh,
                   si0, si1, so0, so1):
    KV = (kv0, kv1)
    SI = (si0, si1)
    SO = (so0, so1)
    core = lax.axis_index("c")
    sub = lax.axis_index("s")
    wid = sub * NC + core
    ones16 = jnp.full((16,), 1.0, _f32)
    zero16 = jnp.zeros((16,), _f32)

    @pl.loop(0, C, step=16)
    def _(g):
        onesv[pl.ds(g, 16)] = ones16

    @pl.loop(0, ELEMS_PER_TILE, step=16)
    def _(g):
        zerov[pl.ds(g, 16)] = zero16

    pltpu.sync_copy(zerov, cnt_sh.at[pl.ds(sub * ELEMS_PER_TILE, ELEMS_PER_TILE)])
    plsc.subcore_barrier()

    for i0 in (0, 1):
        pltpu.async_copy(ckey_hbm.at[pl.ds((wid + i0 * NW) * C, C)],
                         KV[i0], SI[i0])

    @pl.loop(0, PAIRS)
    def _(p):
        for off in (0, 1):
            b = off
            i = 2 * p + off
            cid = wid + i * NW
            cid2 = cid + 2 * NW

            @pl.when(cid < CHUNKS)
            def _():
                pltpu.make_async_copy(ckey_hbm.at[pl.ds(cid * C, C)],
                                      KV[b], SI[b]).wait()
                pltpu.async_copy(onesv, cnt_sh.at[KV[b]], SO[b], add=True)

            @pl.when(cid2 < CHUNKS)
            def _():
                pltpu.make_async_copy(onesv, cnt_sh.at[KV[b]], SO[b]).wait()
                pltpu.async_copy(ckey_hbm.at[pl.ds(cid2 * C, C)], KV[b], SI[b])

    pltpu.make_async_copy(onesv, cnt_sh.at[KV[0]], SO[0]).wait()
    pltpu.make_async_copy(onesv, cnt_sh.at[KV[1]], SO[1]).wait()

    plsc.subcore_barrier()
    pltpu.sync_copy(cnt_sh.at[pl.ds(sub * ELEMS_PER_TILE, ELEMS_PER_TILE)], zerov)
    pltpu.sync_copy(zerov,
                    out_hbm.at[pl.ds(core * KEYS_PAD + sub * ELEMS_PER_TILE,
                                     ELEMS_PER_TILE)])


_sc_count = pl.kernel(
    _sc_count_body,
    out_type=jax.ShapeDtypeStruct((NC * KEYS_PAD,), _f32),
    mesh=_mesh,
    scratch_types=[
        pltpu.VMEM((C,), _i32),
        pltpu.VMEM((C,), _i32),
        pltpu.VMEM((C,), _f32),
        pltpu.VMEM((ELEMS_PER_TILE,), _f32),
        pltpu.VMEM_SHARED((KEYS_PAD,), _f32),
        pltpu.SemaphoreType.DMA,
        pltpu.SemaphoreType.DMA,
        pltpu.SemaphoreType.DMA,
        pltpu.SemaphoreType.DMA,
    ],
    compiler_params=_sc_params,
)


def _sc_main_body(gkey_hbm, ckey_hbm, dst_hbm, z_hbm, cnt_hbm, out_hbm,
                  gk0, gk1, ck0, ck1, cw0, cw1, dv0, dv1, sd0, sd1,
                  sc0, sc1, sw0, sw1, r0, r1,
                  si0, si1, ss0, ss1, sz0, sz1, so0, so1,
                  acc_sh):
    GK = (gk0, gk1)
    CK = (ck0, ck1)
    CW = (cw0, cw1)
    DV = (dv0, dv1)
    SD = (sd0, sd1)
    SCV = (sc0, sc1)
    SCW = (sw0, sw1)
    RW = (r0, r1)
    SI = (si0, si1)
    SS = (ss0, ss1)
    SZ = (sz0, sz1)
    SO = (so0, so1)
    core = lax.axis_index("c")
    sub = lax.axis_index("s")
    wid = sub * NC + core
    zero_row = jnp.zeros((16,), _f32)

    @pl.loop(0, C)
    def _(j):
        for k in range(8):
            r0[j, pl.ds(k * 16, 16)] = zero_row

    for k in range(4):
        pltpu.sync_copy(r0, acc_sh.at[pl.ds(sub * NPT + k * 128, 128)])
    pltpu.sync_copy(r0.at[pl.ds(0, 112)],
                    acc_sh.at[pl.ds(sub * NPT + 512, 112)])

    @pl.when(sub == NS - 1)
    def _():
        pltpu.sync_copy(r0.at[pl.ds(0, 16)], acc_sh.at[pl.ds(NS * NPT, 16)])

    plsc.subcore_barrier()

    def idx_issue(i, b):
        cid = wid + i * NW

        @pl.when(cid < CHUNKS)
        def _():
            base = cid * C
            pltpu.async_copy(gkey_hbm.at[pl.ds(base, C)], GK[b], SI[b])
            pltpu.async_copy(ckey_hbm.at[pl.ds(base, C)], CK[b], SI[b])
            pltpu.async_copy(dst_hbm.at[pl.ds(base, C)], DV[b], SI[b])

    def gathers_issue(i, b):
        cid = wid + i * NW

        @pl.when(cid < CHUNKS)
        def _():
            base = cid * C
            pltpu.make_async_copy(gkey_hbm.at[pl.ds(base, C)], GK[b], SI[b]).wait()
            pltpu.make_async_copy(ckey_hbm.at[pl.ds(base, C)], CK[b], SI[b]).wait()
            pltpu.make_async_copy(dst_hbm.at[pl.ds(base, C)], DV[b], SI[b]).wait()

            @pl.when(i >= 2)
            def _():
                pltpu.make_async_copy(RW[b], acc_sh.at[SD[b]], SO[b]).wait()

            @pl.loop(0, C, step=16)
            def _(g):
                CW[b][pl.ds(g, 16)] = CK[b][pl.ds(g, 16)] + KEYS_PAD

            pltpu.async_copy(cnt_hbm.at[CK[b]], SCV[b], SS[b])
            pltpu.async_copy(cnt_hbm.at[CW[b]], SCW[b], SS[b])
            pltpu.async_copy(z_hbm.at[GK[b]], RW[b], SZ[b])

    def process(i, b):
        cid = wid + i * NW

        @pl.when(cid < CHUNKS)
        def _():
            pltpu.make_async_copy(cnt_hbm.at[CK[b]], SCV[b], SS[b]).wait()
            pltpu.make_async_copy(cnt_hbm.at[CW[b]], SCW[b], SS[b]).wait()
            pltpu.make_async_copy(z_hbm.at[GK[b]], RW[b], SZ[b]).wait()

            @pl.loop(0, C, step=16)
            def _(g):
                cval = SCV[b][pl.ds(g, 16)] + SCW[b][pl.ds(g, 16)]
                SCV[b][pl.ds(g, 16)] = 1.0 / jnp.maximum(cval, 1.0)

            @pl.loop(0, C, step=16)
            def _(g):
                SD[b][pl.ds(g, 16)] = DV[b][pl.ds(g, 16)]

        idx_issue(i + 2, b)

        @pl.when(cid < CHUNKS)
        def _():
            @pl.loop(0, C)
            def _(j):
                jj = lax.broadcast(j, (16,))
                s16 = plsc.load_gather(SCV[b], [jj])
                for k in range(8):
                    RW[b][j, pl.ds(k * 16, 16)] = RW[b][j, pl.ds(k * 16, 16)] * s16

            pltpu.async_copy(RW[b], acc_sh.at[SD[b]], SO[b], add=True)

    idx_issue(0, 0)
    idx_issue(1, 1)
    gathers_issue(0, 0)

    @pl.loop(0, PAIRS)
    def _(p):
        for off in (0, 1):
            i = 2 * p + off
            gathers_issue(i + 1, (off + 1) % 2)
            process(i, off)

    pltpu.make_async_copy(RW[0], acc_sh.at[SD[0]], SO[0]).wait()
    pltpu.make_async_copy(RW[1], acc_sh.at[SD[1]], SO[1]).wait()

    plsc.subcore_barrier()
    for k in range(4):
        pltpu.sync_copy(acc_sh.at[pl.ds(sub * NPT + k * 128, 128)], r0)
        pltpu.sync_copy(r0, out_hbm.at[core, pl.ds(sub * NPT + k * 128, 128)])
    pltpu.sync_copy(acc_sh.at[pl.ds(sub * NPT + 512, 112)], r0.at[pl.ds(0, 112)])
    pltpu.sync_copy(r0.at[pl.ds(0, 112)],
                    out_hbm.at[core, pl.ds(sub * NPT + 512, 112)])

    @pl.when(sub == NS - 1)
    def _():
        pltpu.sync_copy(acc_sh.at[pl.ds(NS * NPT, 16)], r0.at[pl.ds(0, 16)])
        pltpu.sync_copy(r0.at[pl.ds(0, 16)],
                        out_hbm.at[core, pl.ds(NS * NPT, 16)])


_sc_main = pl.kernel(
    _sc_main_body,
    out_type=jax.ShapeDtypeStruct((NC, N, D), _f32),
    mesh=_mesh,
    scratch_types=[
        pltpu.VMEM((C,), _i32),
        pltpu.VMEM((C,), _i32),
        pltpu.VMEM((C,), _i32),
        pltpu.VMEM((C,), _i32),
        pltpu.VMEM((C,), _i32),
        pltpu.VMEM((C,), _i32),
        pltpu.VMEM((C,), _i32),
        pltpu.VMEM((C,), _i32),
        pltpu.VMEM((C,), _i32),
        pltpu.VMEM((C,), _i32),
        pltpu.VMEM((C,), _f32),
        pltpu.VMEM((C,), _f32),
        pltpu.VMEM((C,), _f32),
        pltpu.VMEM((C,), _f32),
        pltpu.VMEM((C, D), _f32),
        pltpu.VMEM((C, D), _f32),
        pltpu.SemaphoreType.DMA,
        pltpu.SemaphoreType.DMA,
        pltpu.SemaphoreType.DMA,
        pltpu.SemaphoreType.DMA,
        pltpu.SemaphoreType.DMA,
        pltpu.SemaphoreType.DMA,
        pltpu.SemaphoreType.DMA,
        pltpu.SemaphoreType.DMA,
        pltpu.VMEM_SHARED((N, D), _f32),
    ],
    compiler_params=_sc_params,
)


def _tc_final_body(part_ref, root_ref, out_ref):
    p = part_ref[...]
    out_ref[...] = p[0] + p[1] + root_ref[...]


def _tc_final(parts, root):
    nb = 10
    bn = N // nb
    return pl.pallas_call(
        _tc_final_body,
        grid=(nb,),
        in_specs=[
            pl.BlockSpec((NC, bn, D), lambda i: (0, i, 0)),
            pl.BlockSpec((bn, D), lambda i: (i, 0)),
        ],
        out_specs=pl.BlockSpec((bn, D), lambda i: (i, 0)),
        out_shape=jax.ShapeDtypeStruct((N, D), _f32),
    )(parts, root)


def kernel(x_src, x_target, edge_index, edge_type, target_node_type,
           src_node_type, W_rel, W_root, b_root):
    src2d = edge_index[0].astype(_i32).reshape(CHUNKS, C)
    dst2d = edge_index[1].astype(_i32).reshape(CHUNKS, C)
    typ2d = edge_type.astype(_i32).reshape(CHUNKS, C)
    tnt2d = target_node_type.astype(_i32).reshape(N, 1)

    gkey2d, ckey2d = _tc_keys(src2d, dst2d, typ2d)
    gkey = gkey2d.reshape(E)
    ckey = ckey2d.reshape(E)
    dst = dst2d.reshape(E)

    cnt = _sc_count(ckey)

    z, root = _tc_precompute(
        x_src.astype(_f32), x_target.astype(_f32), W_rel.astype(_f32),
        W_root.astype(_f32), b_root.astype(_f32), tnt2d)
    z = z.reshape(KEYS, D)

    parts = _sc_main(gkey, ckey, dst, z, cnt)
    return _tc_final(parts, root)


# speedup vs baseline: 1.3301x; 1.1703x over previous
import dataclasses
import functools

import jax
import jax.numpy as jnp
from jax import lax
from jax.experimental import pallas as pl
from jax.experimental.pallas import tpu as pltpu
from jax.experimental.pallas import tpu_sc as plsc

N = 10000
E = 320000
D = 128
R = 7
T = 4
KEYS = R * N
KEYS_PAD = 70144
C = 128
CHUNKS = E // C
NC = 2
NS = 16
NW = NC * NS
ITERS = (CHUNKS + NW - 1) // NW
PAIRS = (ITERS + 1) // 2
ITERS_CNT = (CHUNKS + NS - 1) // NS
PAIRS_CNT = (ITERS_CNT + 1) // 2
ELEMS_PER_TILE = KEYS_PAD // NS
NPT = 624

_mesh = plsc.VectorSubcoreMesh(core_axis_name="c", subcore_axis_name="s")
_f32 = jnp.float32
_i32 = jnp.int32

_sc_params = pltpu.CompilerParams()
if "needs_layout_passes" in pltpu.CompilerParams.__dataclass_fields__:
    _sc_params = dataclasses.replace(_sc_params, needs_layout_passes=False)


def _tc_pre_body(xs_ref, xt_ref, wcat_ref, wrcat_ref, broot_ref, tnt_ref,
                 z_ref, root_ref):
    z_ref[...] = jnp.dot(xs_ref[...], wcat_ref[...],
                         precision=lax.Precision.HIGHEST,
                         preferred_element_type=_f32)
    rall = jnp.dot(xt_ref[...], wrcat_ref[...],
                   precision=lax.Precision.HIGHEST,
                   preferred_element_type=_f32)
    tt = tnt_ref[...]
    acc = jnp.zeros((tt.shape[0], D), _f32)
    for i in range(T):
        v = rall[:, i * D:(i + 1) * D] + broot_ref[i][None, :]
        acc = acc + jnp.where(tt == i, 1.0, 0.0).astype(_f32) * v
    root_ref[...] = acc


def _tc_precompute(x_src, x_target, Wcat, Wrcat, b_root, tnt2d):
    nb = 10
    bn = N // nb
    return pl.pallas_call(
        _tc_pre_body,
        grid=(nb,),
        in_specs=[
            pl.BlockSpec((bn, D), lambda i: (i, 0)),
            pl.BlockSpec((bn, D), lambda i: (i, 0)),
            pl.BlockSpec((D, R * D), lambda i: (0, 0)),
            pl.BlockSpec((D, T * D), lambda i: (0, 0)),
            pl.BlockSpec((T, D), lambda i: (0, 0)),
            pl.BlockSpec((bn, 1), lambda i: (i, 0)),
        ],
        out_specs=[
            pl.BlockSpec((bn, R * D), lambda i: (i, 0)),
            pl.BlockSpec((bn, D), lambda i: (i, 0)),
        ],
        out_shape=[
            jax.ShapeDtypeStruct((N, R * D), _f32),
            jax.ShapeDtypeStruct((N, D), _f32),
        ],
        compiler_params=pltpu.CompilerParams(
            dimension_semantics=("parallel",)),
    )(x_src, x_target, Wcat, Wrcat, b_root, tnt2d)


def _tc_keys_body(ei_ref, typ_ref, gk_ref, ck_ref):
    t = typ_ref[...]
    gk_ref[...] = ei_ref[0] * R + t
    ck_ref[...] = ei_ref[1] * R + t


def _tc_keys(edge_index, edge_type):
    return pl.pallas_call(
        _tc_keys_body,
        out_shape=[jax.ShapeDtypeStruct((E,), _i32)] * 2,
    )(edge_index, edge_type)


def _sc_count_body(ckey_hbm, out_hbm, kv0, kv1, onesv, zerov, cnt_sh,
                   si0, si1, so0, so1):
    KV = (kv0, kv1)
    SI = (si0, si1)
    SO = (so0, so1)
    core = lax.axis_index("c")
    sub = lax.axis_index("s")
    wid = sub * NC + core
    ones16 = jnp.full((16,), 1.0, _f32)
    zero16 = jnp.zeros((16,), _f32)

    @pl.loop(0, C, step=16)
    def _(g):
        onesv[pl.ds(g, 16)] = ones16

    @pl.loop(0, ELEMS_PER_TILE, step=16)
    def _(g):
        zerov[pl.ds(g, 16)] = zero16

    pltpu.sync_copy(zerov, cnt_sh.at[pl.ds(sub * ELEMS_PER_TILE, ELEMS_PER_TILE)])
    plsc.subcore_barrier()

    for i0 in (0, 1):
        pltpu.async_copy(ckey_hbm.at[pl.ds((wid + i0 * NW) * C, C)],
                         KV[i0], SI[i0])

    @pl.loop(0, PAIRS)
    def _(p):
        for off in (0, 1):
            b = off
            i = 2 * p + off
            cid = wid + i * NW
            cid2 = cid + 2 * NW

            @pl.when(cid < CHUNKS)
            def _():
                pltpu.make_async_copy(ckey_hbm.at[pl.ds(cid * C, C)],
                                      KV[b], SI[b]).wait()
                pltpu.async_copy(onesv, cnt_sh.at[KV[b]], SO[b], add=True)

            @pl.when(cid2 < CHUNKS)
            def _():
                pltpu.make_async_copy(onesv, cnt_sh.at[KV[b]], SO[b]).wait()
                pltpu.async_copy(ckey_hbm.at[pl.ds(cid2 * C, C)], KV[b], SI[b])

    pltpu.make_async_copy(onesv, cnt_sh.at[KV[0]], SO[0]).wait()
    pltpu.make_async_copy(onesv, cnt_sh.at[KV[1]], SO[1]).wait()

    plsc.subcore_barrier()
    pltpu.sync_copy(cnt_sh.at[pl.ds(sub * ELEMS_PER_TILE, ELEMS_PER_TILE)], zerov)
    pltpu.sync_copy(zerov,
                    out_hbm.at[pl.ds(core * KEYS_PAD + sub * ELEMS_PER_TILE,
                                     ELEMS_PER_TILE)])


_sc_count = pl.kernel(
    _sc_count_body,
    out_type=jax.ShapeDtypeStruct((NC * KEYS_PAD,), _f32),
    mesh=_mesh,
    scratch_types=[
        pltpu.VMEM((C,), _i32),
        pltpu.VMEM((C,), _i32),
        pltpu.VMEM((C,), _f32),
        pltpu.VMEM((ELEMS_PER_TILE,), _f32),
        pltpu.VMEM_SHARED((KEYS_PAD,), _f32),
        pltpu.SemaphoreType.DMA,
        pltpu.SemaphoreType.DMA,
        pltpu.SemaphoreType.DMA,
        pltpu.SemaphoreType.DMA,
    ],
    compiler_params=_sc_params,
)


def _sc_main_body(gkey_hbm, ckey_hbm, ei_hbm, z_hbm, cnt_hbm, out_hbm,
                  gk0, gk1, ck0, ck1, cw0, cw1, dv0, dv1, sd0, sd1,
                  sc0, sc1, sw0, sw1, r0, r1,
                  si0, si1, ss0, ss1, sz0, sz1, so0, so1,
                  acc_sh):
    GK = (gk0, gk1)
    CK = (ck0, ck1)
    CW = (cw0, cw1)
    DV = (dv0, dv1)
    SD = (sd0, sd1)
    SCV = (sc0, sc1)
    SCW = (sw0, sw1)
    RW = (r0, r1)
    SI = (si0, si1)
    SS = (ss0, ss1)
    SZ = (sz0, sz1)
    SO = (so0, so1)
    core = lax.axis_index("c")
    sub = lax.axis_index("s")
    wid = sub * NC + core
    zero_row = jnp.zeros((16,), _f32)

    @pl.loop(0, C)
    def _(j):
        for k in range(8):
            r0[j, pl.ds(k * 16, 16)] = zero_row

    for k in range(4):
        pltpu.sync_copy(r0, acc_sh.at[pl.ds(sub * NPT + k * 128, 128)])
    pltpu.sync_copy(r0.at[pl.ds(0, 112)],
                    acc_sh.at[pl.ds(sub * NPT + 512, 112)])

    @pl.when(sub == NS - 1)
    def _():
        pltpu.sync_copy(r0.at[pl.ds(0, 16)], acc_sh.at[pl.ds(NS * NPT, 16)])

    plsc.subcore_barrier()

    def idx_issue(i, b):
        cid = wid + i * NW

        @pl.when(cid < CHUNKS)
        def _():
            base = cid * C
            pltpu.async_copy(gkey_hbm.at[pl.ds(base, C)], GK[b], SI[b])
            pltpu.async_copy(ckey_hbm.at[pl.ds(base, C)], CK[b], SI[b])
            pltpu.async_copy(ei_hbm.at[1, pl.ds(base, C)], DV[b], SI[b])

    def gathers_issue(i, b):
        cid = wid + i * NW

        @pl.when(cid < CHUNKS)
        def _():
            base = cid * C
            pltpu.make_async_copy(gkey_hbm.at[pl.ds(base, C)], GK[b], SI[b]).wait()
            pltpu.make_async_copy(ckey_hbm.at[pl.ds(base, C)], CK[b], SI[b]).wait()
            pltpu.make_async_copy(ei_hbm.at[1, pl.ds(base, C)], DV[b], SI[b]).wait()

            @pl.when(i >= 2)
            def _():
                pltpu.make_async_copy(RW[b], acc_sh.at[SD[b]], SO[b]).wait()

            @pl.loop(0, C, step=16)
            def _(g):
                CW[b][pl.ds(g, 16)] = CK[b][pl.ds(g, 16)] + KEYS_PAD

            pltpu.async_copy(cnt_hbm.at[CK[b]], SCV[b], SS[b])
            pltpu.async_copy(cnt_hbm.at[CW[b]], SCW[b], SS[b])
            pltpu.async_copy(z_hbm.at[GK[b]], RW[b], SZ[b])

    def process(i, b):
        cid = wid + i * NW

        @pl.when(cid < CHUNKS)
        def _():
            pltpu.make_async_copy(cnt_hbm.at[CK[b]], SCV[b], SS[b]).wait()
            pltpu.make_async_copy(cnt_hbm.at[CW[b]], SCW[b], SS[b]).wait()
            pltpu.make_async_copy(z_hbm.at[GK[b]], RW[b], SZ[b]).wait()

            @pl.loop(0, C, step=16)
            def _(g):
                cval = SCV[b][pl.ds(g, 16)] + SCW[b][pl.ds(g, 16)]
                SCV[b][pl.ds(g, 16)] = 1.0 / jnp.maximum(cval, 1.0)

            @pl.loop(0, C, step=16)
            def _(g):
                SD[b][pl.ds(g, 16)] = DV[b][pl.ds(g, 16)]

        idx_issue(i + 2, b)

        @pl.when(cid < CHUNKS)
        def _():
            @pl.loop(0, C, step=2)
            def _(j):
                s16a = plsc.load_gather(SCV[b], [lax.broadcast(j, (16,))])
                s16b = plsc.load_gather(SCV[b], [lax.broadcast(j + 1, (16,))])
                for k in range(8):
                    RW[b][j, pl.ds(k * 16, 16)] = RW[b][j, pl.ds(k * 16, 16)] * s16a
                for k in range(8):
                    RW[b][j + 1, pl.ds(k * 16, 16)] = (
                        RW[b][j + 1, pl.ds(k * 16, 16)] * s16b)

            pltpu.async_copy(RW[b], acc_sh.at[SD[b]], SO[b], add=True)

    idx_issue(0, 0)
    idx_issue(1, 1)
    gathers_issue(0, 0)

    @pl.loop(0, PAIRS)
    def _(p):
        for off in (0, 1):
            i = 2 * p + off
            gathers_issue(i + 1, (off + 1) % 2)
            process(i, off)

    pltpu.make_async_copy(RW[0], acc_sh.at[SD[0]], SO[0]).wait()
    pltpu.make_async_copy(RW[1], acc_sh.at[SD[1]], SO[1]).wait()

    plsc.subcore_barrier()
    for k in range(4):
        pltpu.sync_copy(acc_sh.at[pl.ds(sub * NPT + k * 128, 128)], r0)
        pltpu.sync_copy(r0, out_hbm.at[core, pl.ds(sub * NPT + k * 128, 128)])
    pltpu.sync_copy(acc_sh.at[pl.ds(sub * NPT + 512, 112)], r0.at[pl.ds(0, 112)])
    pltpu.sync_copy(r0.at[pl.ds(0, 112)],
                    out_hbm.at[core, pl.ds(sub * NPT + 512, 112)])

    @pl.when(sub == NS - 1)
    def _():
        pltpu.sync_copy(acc_sh.at[pl.ds(NS * NPT, 16)], r0.at[pl.ds(0, 16)])
        pltpu.sync_copy(r0.at[pl.ds(0, 16)],
                        out_hbm.at[core, pl.ds(NS * NPT, 16)])


_sc_main = pl.kernel(
    _sc_main_body,
    out_type=jax.ShapeDtypeStruct((NC, N, D), _f32),
    mesh=_mesh,
    scratch_types=[
        pltpu.VMEM((C,), _i32),
        pltpu.VMEM((C,), _i32),
        pltpu.VMEM((C,), _i32),
        pltpu.VMEM((C,), _i32),
        pltpu.VMEM((C,), _i32),
        pltpu.VMEM((C,), _i32),
        pltpu.VMEM((C,), _i32),
        pltpu.VMEM((C,), _i32),
        pltpu.VMEM((C,), _i32),
        pltpu.VMEM((C,), _i32),
        pltpu.VMEM((C,), _f32),
        pltpu.VMEM((C,), _f32),
        pltpu.VMEM((C,), _f32),
        pltpu.VMEM((C,), _f32),
        pltpu.VMEM((C, D), _f32),
        pltpu.VMEM((C, D), _f32),
        pltpu.SemaphoreType.DMA,
        pltpu.SemaphoreType.DMA,
        pltpu.SemaphoreType.DMA,
        pltpu.SemaphoreType.DMA,
        pltpu.SemaphoreType.DMA,
        pltpu.SemaphoreType.DMA,
        pltpu.SemaphoreType.DMA,
        pltpu.SemaphoreType.DMA,
        pltpu.VMEM_SHARED((N, D), _f32),
    ],
    compiler_params=_sc_params,
)


def _tc_final_body(part_ref, root_ref, out_ref):
    p = part_ref[...]
    out_ref[...] = p[0] + p[1] + root_ref[...]


def _tc_final(parts, root):
    nb = 10
    bn = N // nb
    return pl.pallas_call(
        _tc_final_body,
        grid=(nb,),
        in_specs=[
            pl.BlockSpec((NC, bn, D), lambda i: (0, i, 0)),
            pl.BlockSpec((bn, D), lambda i: (i, 0)),
        ],
        out_specs=pl.BlockSpec((bn, D), lambda i: (i, 0)),
        out_shape=jax.ShapeDtypeStruct((N, D), _f32),
    )(parts, root)


def kernel(x_src, x_target, edge_index, edge_type, target_node_type,
           src_node_type, W_rel, W_root, b_root):
    edge_index = edge_index.astype(_i32)
    edge_type = edge_type.astype(_i32)
    tnt2d = target_node_type.astype(_i32).reshape(N, 1)
    Wcat = jnp.transpose(W_rel.astype(_f32), (2, 0, 1)).reshape(D, R * D)
    Wrcat = jnp.transpose(W_root.astype(_f32), (2, 0, 1)).reshape(D, T * D)

    gkey, ckey = _tc_keys(edge_index, edge_type)
    cnt = _sc_count(ckey)

    z, root = _tc_precompute(x_src.astype(_f32), x_target.astype(_f32),
                             Wcat, Wrcat, b_root.astype(_f32), tnt2d)
    z = z.reshape(KEYS, D)

    parts = _sc_main(gkey, ckey, edge_index, z, cnt)
    return _tc_final(parts, root)


# speedup vs baseline: 1.5187x; 1.1418x over previous
import dataclasses
import functools

import jax
import jax.numpy as jnp
from jax import lax
from jax.experimental import pallas as pl
from jax.experimental.pallas import tpu as pltpu
from jax.experimental.pallas import tpu_sc as plsc

N = 10000
E = 320000
D = 128
R = 7
T = 4
KEYS = R * N
KEYS_PAD = 70144
C = 128
CHUNKS = E // C
NC = 2
NS = 16
NW = NC * NS
ITERS = (CHUNKS + NW - 1) // NW
PAIRS = (ITERS + 1) // 2
ITERS_CNT = (CHUNKS + NS - 1) // NS
PAIRS_CNT = (ITERS_CNT + 1) // 2
ELEMS_PER_TILE = KEYS_PAD // NS
NPT = 624

_mesh = plsc.VectorSubcoreMesh(core_axis_name="c", subcore_axis_name="s")
_f32 = jnp.float32
_i32 = jnp.int32

_sc_params = pltpu.CompilerParams()
if "needs_layout_passes" in pltpu.CompilerParams.__dataclass_fields__:
    _sc_params = dataclasses.replace(_sc_params, needs_layout_passes=False)


def _tc_pre_body(xs_ref, xt_ref, wcat_ref, wrcat_ref, broot_ref, tnt_ref,
                 z_ref, root_ref):
    z2 = jnp.dot(xs_ref[...], wcat_ref[...],
                 precision=lax.Precision.HIGHEST,
                 preferred_element_type=_f32)
    for r in range(R):
        z_ref[r] = z2[:, r * D:(r + 1) * D]
    rall = jnp.dot(xt_ref[...], wrcat_ref[...],
                   precision=lax.Precision.HIGHEST,
                   preferred_element_type=_f32)
    tt = tnt_ref[...]
    acc = jnp.zeros((tt.shape[0], D), _f32)
    for i in range(T):
        v = rall[:, i * D:(i + 1) * D] + broot_ref[i][None, :]
        acc = acc + jnp.where(tt == i, 1.0, 0.0).astype(_f32) * v
    root_ref[...] = acc


def _tc_precompute(x_src, x_target, Wcat, Wrcat, b_root, tnt2d):
    nb = 10
    bn = N // nb
    return pl.pallas_call(
        _tc_pre_body,
        grid=(nb,),
        in_specs=[
            pl.BlockSpec((bn, D), lambda i: (i, 0)),
            pl.BlockSpec((bn, D), lambda i: (i, 0)),
            pl.BlockSpec((D, R * D), lambda i: (0, 0)),
            pl.BlockSpec((D, T * D), lambda i: (0, 0)),
            pl.BlockSpec((T, D), lambda i: (0, 0)),
            pl.BlockSpec((bn, 1), lambda i: (i, 0)),
        ],
        out_specs=[
            pl.BlockSpec((R, bn, D), lambda i: (0, i, 0)),
            pl.BlockSpec((bn, D), lambda i: (i, 0)),
        ],
        out_shape=[
            jax.ShapeDtypeStruct((R, N, D), _f32),
            jax.ShapeDtypeStruct((N, D), _f32),
        ],
        compiler_params=pltpu.CompilerParams(
            dimension_semantics=("parallel",)),
    )(x_src, x_target, Wcat, Wrcat, b_root, tnt2d)


def _tc_keys_body(ei_ref, typ_ref, pk_ref, ck_ref):
    t = typ_ref[...]
    gk = t * N + ei_ref[0]
    ck = t * N + ei_ref[1]
    ck_ref[...] = ck
    pk_ref[...] = jnp.concatenate(
        [gk.reshape(CHUNKS, C), ck.reshape(CHUNKS, C),
         (ck + KEYS_PAD).reshape(CHUNKS, C),
         ei_ref[1].reshape(CHUNKS, C)], axis=1)


def _tc_keys(edge_index, edge_type):
    return pl.pallas_call(
        _tc_keys_body,
        out_shape=[jax.ShapeDtypeStruct((CHUNKS, 4 * C), _i32),
                   jax.ShapeDtypeStruct((E,), _i32)],
    )(edge_index, edge_type)


def _sc_count_body(ckey_hbm, out_hbm, kv0, kv1, onesv, zerov, cnt_sh,
                   si0, si1, so0, so1):
    KV = (kv0, kv1)
    SI = (si0, si1)
    SO = (so0, so1)
    core = lax.axis_index("c")
    sub = lax.axis_index("s")
    wid = sub * NC + core
    ones16 = jnp.full((16,), 1.0, _f32)
    zero16 = jnp.zeros((16,), _f32)

    @pl.loop(0, C, step=16)
    def _(g):
        onesv[pl.ds(g, 16)] = ones16

    @pl.loop(0, ELEMS_PER_TILE, step=16)
    def _(g):
        zerov[pl.ds(g, 16)] = zero16

    pltpu.sync_copy(zerov, cnt_sh.at[pl.ds(sub * ELEMS_PER_TILE, ELEMS_PER_TILE)])
    plsc.subcore_barrier()

    for i0 in (0, 1):
        pltpu.async_copy(ckey_hbm.at[pl.ds((wid + i0 * NW) * C, C)],
                         KV[i0], SI[i0])

    @pl.loop(0, PAIRS)
    def _(p):
        for off in (0, 1):
            b = off
            i = 2 * p + off
            cid = wid + i * NW
            cid2 = cid + 2 * NW

            @pl.when(cid < CHUNKS)
            def _():
                pltpu.make_async_copy(ckey_hbm.at[pl.ds(cid * C, C)],
                                      KV[b], SI[b]).wait()
                pltpu.async_copy(onesv, cnt_sh.at[KV[b]], SO[b], add=True)

            @pl.when(cid2 < CHUNKS)
            def _():
                pltpu.make_async_copy(onesv, cnt_sh.at[KV[b]], SO[b]).wait()
                pltpu.async_copy(ckey_hbm.at[pl.ds(cid2 * C, C)], KV[b], SI[b])

    pltpu.make_async_copy(onesv, cnt_sh.at[KV[0]], SO[0]).wait()
    pltpu.make_async_copy(onesv, cnt_sh.at[KV[1]], SO[1]).wait()

    plsc.subcore_barrier()
    pltpu.sync_copy(cnt_sh.at[pl.ds(sub * ELEMS_PER_TILE, ELEMS_PER_TILE)], zerov)
    pltpu.sync_copy(zerov,
                    out_hbm.at[pl.ds(core * KEYS_PAD + sub * ELEMS_PER_TILE,
                                     ELEMS_PER_TILE)])


_sc_count = pl.kernel(
    _sc_count_body,
    out_type=jax.ShapeDtypeStruct((NC * KEYS_PAD,), _f32),
    mesh=_mesh,
    scratch_types=[
        pltpu.VMEM((C,), _i32),
        pltpu.VMEM((C,), _i32),
        pltpu.VMEM((C,), _f32),
        pltpu.VMEM((ELEMS_PER_TILE,), _f32),
        pltpu.VMEM_SHARED((KEYS_PAD,), _f32),
        pltpu.SemaphoreType.DMA,
        pltpu.SemaphoreType.DMA,
        pltpu.SemaphoreType.DMA,
        pltpu.SemaphoreType.DMA,
    ],
    compiler_params=_sc_params,
)


def _sc_main_body(pk_hbm, z_hbm, cnt_hbm, out_hbm,
                  kb0, kb1, sd0, sd1,
                  sc0, sc1, sw0, sw1, r0, r1,
                  si0, si1, ss0, ss1, sz0, sz1, so0, so1,
                  acc_sh):
    KB = (kb0, kb1)
    SD = (sd0, sd1)
    SCV = (sc0, sc1)
    SCW = (sw0, sw1)
    RW = (r0, r1)
    SI = (si0, si1)
    SS = (ss0, ss1)
    SZ = (sz0, sz1)
    SO = (so0, so1)
    core = lax.axis_index("c")
    sub = lax.axis_index("s")
    wid = sub * NC + core
    zero_row = jnp.zeros((16,), _f32)

    @pl.loop(0, C)
    def _(j):
        for k in range(8):
            r0[j, pl.ds(k * 16, 16)] = zero_row

    for k in range(4):
        pltpu.sync_copy(r0, acc_sh.at[pl.ds(sub * NPT + k * 128, 128)])
    pltpu.sync_copy(r0.at[pl.ds(0, 112)],
                    acc_sh.at[pl.ds(sub * NPT + 512, 112)])

    @pl.when(sub == NS - 1)
    def _():
        pltpu.sync_copy(r0.at[pl.ds(0, 16)], acc_sh.at[pl.ds(NS * NPT, 16)])

    plsc.subcore_barrier()

    def idx_issue(i, b):
        cid = wid + i * NW

        @pl.when(cid < CHUNKS)
        def _():
            pltpu.async_copy(pk_hbm.at[cid], KB[b], SI[b])

    def gathers_issue(i, b):
        cid = wid + i * NW

        @pl.when(cid < CHUNKS)
        def _():
            pltpu.make_async_copy(pk_hbm.at[cid], KB[b], SI[b]).wait()

            @pl.when(i >= 2)
            def _():
                pltpu.make_async_copy(RW[b], acc_sh.at[SD[b]], SO[b]).wait()

            pltpu.async_copy(cnt_hbm.at[KB[b].at[pl.ds(C, C)]], SCV[b], SS[b])
            pltpu.async_copy(cnt_hbm.at[KB[b].at[pl.ds(2 * C, C)]], SCW[b], SS[b])
            pltpu.async_copy(z_hbm.at[KB[b].at[pl.ds(0, C)]], RW[b], SZ[b])

    def process(i, b):
        cid = wid + i * NW

        @pl.when(cid < CHUNKS)
        def _():
            pltpu.make_async_copy(cnt_hbm.at[KB[b].at[pl.ds(C, C)]],
                                  SCV[b], SS[b]).wait()
            pltpu.make_async_copy(cnt_hbm.at[KB[b].at[pl.ds(2 * C, C)]],
                                  SCW[b], SS[b]).wait()
            pltpu.make_async_copy(z_hbm.at[KB[b].at[pl.ds(0, C)]],
                                  RW[b], SZ[b]).wait()

            @pl.loop(0, C, step=16)
            def _(g):
                cval = SCV[b][pl.ds(g, 16)] + SCW[b][pl.ds(g, 16)]
                SCV[b][pl.ds(g, 16)] = 1.0 / jnp.maximum(cval, 1.0)

            @pl.loop(0, C, step=16)
            def _(g):
                SD[b][pl.ds(g, 16)] = KB[b][pl.ds(3 * C + g, 16)]

        idx_issue(i + 2, b)

        @pl.when(cid < CHUNKS)
        def _():
            @pl.loop(0, C, step=2)
            def _(j):
                s16a = plsc.load_gather(SCV[b], [lax.broadcast(j, (16,))])
                s16b = plsc.load_gather(SCV[b], [lax.broadcast(j + 1, (16,))])
                for k in range(8):
                    RW[b][j, pl.ds(k * 16, 16)] = RW[b][j, pl.ds(k * 16, 16)] * s16a
                for k in range(8):
                    RW[b][j + 1, pl.ds(k * 16, 16)] = (
                        RW[b][j + 1, pl.ds(k * 16, 16)] * s16b)

            pltpu.async_copy(RW[b], acc_sh.at[SD[b]], SO[b], add=True)

    idx_issue(0, 0)
    idx_issue(1, 1)
    gathers_issue(0, 0)

    @pl.loop(0, PAIRS)
    def _(p):
        for off in (0, 1):
            i = 2 * p + off
            gathers_issue(i + 1, (off + 1) % 2)
            process(i, off)

    pltpu.make_async_copy(RW[0], acc_sh.at[SD[0]], SO[0]).wait()
    pltpu.make_async_copy(RW[1], acc_sh.at[SD[1]], SO[1]).wait()

    plsc.subcore_barrier()
    for k in range(4):
        pltpu.sync_copy(acc_sh.at[pl.ds(sub * NPT + k * 128, 128)], r0)
        pltpu.sync_copy(r0, out_hbm.at[core, pl.ds(sub * NPT + k * 128, 128)])
    pltpu.sync_copy(acc_sh.at[pl.ds(sub * NPT + 512, 112)], r0.at[pl.ds(0, 112)])
    pltpu.sync_copy(r0.at[pl.ds(0, 112)],
                    out_hbm.at[core, pl.ds(sub * NPT + 512, 112)])

    @pl.when(sub == NS - 1)
    def _():
        pltpu.sync_copy(acc_sh.at[pl.ds(NS * NPT, 16)], r0.at[pl.ds(0, 16)])
        pltpu.sync_copy(r0.at[pl.ds(0, 16)],
                        out_hbm.at[core, pl.ds(NS * NPT, 16)])


_sc_main = pl.kernel(
    _sc_main_body,
    out_type=jax.ShapeDtypeStruct((NC, N, D), _f32),
    mesh=_mesh,
    scratch_types=[
        pltpu.VMEM((4 * C,), _i32),
        pltpu.VMEM((4 * C,), _i32),
        pltpu.VMEM((C,), _i32),
        pltpu.VMEM((C,), _i32),
        pltpu.VMEM((C,), _f32),
        pltpu.VMEM((C,), _f32),
        pltpu.VMEM((C,), _f32),
        pltpu.VMEM((C,), _f32),
        pltpu.VMEM((C, D), _f32),
        pltpu.VMEM((C, D), _f32),
        pltpu.SemaphoreType.DMA,
        pltpu.SemaphoreType.DMA,
        pltpu.SemaphoreType.DMA,
        pltpu.SemaphoreType.DMA,
        pltpu.SemaphoreType.DMA,
        pltpu.SemaphoreType.DMA,
        pltpu.SemaphoreType.DMA,
        pltpu.SemaphoreType.DMA,
        pltpu.VMEM_SHARED((N, D), _f32),
    ],
    compiler_params=_sc_params,
)


def _tc_final_body(part_ref, root_ref, out_ref):
    p = part_ref[...]
    out_ref[...] = p[0] + p[1] + root_ref[...]


def _tc_final(parts, root):
    nb = 10
    bn = N // nb
    return pl.pallas_call(
        _tc_final_body,
        grid=(nb,),
        in_specs=[
            pl.BlockSpec((NC, bn, D), lambda i: (0, i, 0)),
            pl.BlockSpec((bn, D), lambda i: (i, 0)),
        ],
        out_specs=pl.BlockSpec((bn, D), lambda i: (i, 0)),
        out_shape=jax.ShapeDtypeStruct((N, D), _f32),
    )(parts, root)


def kernel(x_src, x_target, edge_index, edge_type, target_node_type,
           src_node_type, W_rel, W_root, b_root):
    edge_index = edge_index.astype(_i32)
    edge_type = edge_type.astype(_i32)
    tnt2d = target_node_type.astype(_i32).reshape(N, 1)
    Wcat = jnp.transpose(W_rel.astype(_f32), (2, 0, 1)).reshape(D, R * D)
    Wrcat = jnp.transpose(W_root.astype(_f32), (2, 0, 1)).reshape(D, T * D)

    packed, ckey = _tc_keys(edge_index, edge_type)
    cnt = _sc_count(ckey)

    z, root = _tc_precompute(x_src.astype(_f32), x_target.astype(_f32),
                             Wcat, Wrcat, b_root.astype(_f32), tnt2d)
    z = z.reshape(KEYS, D)

    parts = _sc_main(packed, z, cnt)
    return _tc_final(parts, root)


# speedup vs baseline: 1.7442x; 1.1485x over previous
import dataclasses
import functools

import jax
import jax.numpy as jnp
from jax import lax
from jax.experimental import pallas as pl
from jax.experimental.pallas import tpu as pltpu
from jax.experimental.pallas import tpu_sc as plsc

N = 10000
E = 320000
D = 128
R = 7
T = 4
KEYS = R * N
KEYS_PAD = 70144
C = 128
CHUNKS = E // C
NC = 2
NS = 16
NW = NC * NS
ITERS = (CHUNKS + NW - 1) // NW
PAIRS = (ITERS + 1) // 2
ITERS_CNT = (CHUNKS + NS - 1) // NS
PAIRS_CNT = (ITERS_CNT + 1) // 2
ELEMS_PER_TILE = KEYS_PAD // NS
NPT = 624

_mesh = plsc.VectorSubcoreMesh(core_axis_name="c", subcore_axis_name="s")
_f32 = jnp.float32
_i32 = jnp.int32

_sc_params = pltpu.CompilerParams()
if "needs_layout_passes" in pltpu.CompilerParams.__dataclass_fields__:
    _sc_params = dataclasses.replace(_sc_params, needs_layout_passes=False)


def _tc_pre_body(xs_ref, xt_ref, wcat_ref, wrcat_ref, broot_ref, tnt_ref,
                 z_ref, root_ref):
    z2 = jnp.dot(xs_ref[...], wcat_ref[...], preferred_element_type=_f32)
    for r in range(R):
        z_ref[r] = z2[:, r * D:(r + 1) * D]
    rall = jnp.dot(xt_ref[...], wrcat_ref[...], preferred_element_type=_f32)
    tt = tnt_ref[...]
    acc = jnp.zeros((tt.shape[0], D), _f32)
    for i in range(T):
        v = rall[:, i * D:(i + 1) * D] + broot_ref[i][None, :]
        acc = acc + jnp.where(tt == i, 1.0, 0.0).astype(_f32) * v
    root_ref[...] = acc


def _tc_precompute(x_src, x_target, Wcat, Wrcat, b_root, tnt2d):
    nb = 5
    bn = N // nb
    return pl.pallas_call(
        _tc_pre_body,
        grid=(nb,),
        in_specs=[
            pl.BlockSpec((bn, D), lambda i: (i, 0)),
            pl.BlockSpec((bn, D), lambda i: (i, 0)),
            pl.BlockSpec((D, R * D), lambda i: (0, 0)),
            pl.BlockSpec((D, T * D), lambda i: (0, 0)),
            pl.BlockSpec((T, D), lambda i: (0, 0)),
            pl.BlockSpec((bn, 1), lambda i: (i, 0)),
        ],
        out_specs=[
            pl.BlockSpec((R, bn, D), lambda i: (0, i, 0)),
            pl.BlockSpec((bn, D), lambda i: (i, 0)),
        ],
        out_shape=[
            jax.ShapeDtypeStruct((R, N, D), _f32),
            jax.ShapeDtypeStruct((N, D), _f32),
        ],
        compiler_params=pltpu.CompilerParams(
            dimension_semantics=("parallel",)),
    )(x_src, x_target, Wcat, Wrcat, b_root, tnt2d)


def _tc_keys_body(ei_ref, typ_ref, pk_ref, ck_ref):
    t = typ_ref[...]
    gk = t * N + ei_ref[0]
    ck = t * N + ei_ref[1]
    ck_ref[...] = ck
    pk_ref[...] = jnp.concatenate(
        [gk.reshape(CHUNKS, C), ck.reshape(CHUNKS, C),
         (ck + KEYS_PAD).reshape(CHUNKS, C),
         ei_ref[1].reshape(CHUNKS, C)], axis=1)


def _tc_keys(edge_index, edge_type):
    return pl.pallas_call(
        _tc_keys_body,
        out_shape=[jax.ShapeDtypeStruct((CHUNKS, 4 * C), _i32),
                   jax.ShapeDtypeStruct((E,), _i32)],
    )(edge_index, edge_type)


def _sc_count_body(ckey_hbm, out_hbm, kv0, kv1, onesv, zerov, cnt_sh,
                   si0, si1, so0, so1):
    KV = (kv0, kv1)
    SI = (si0, si1)
    SO = (so0, so1)
    core = lax.axis_index("c")
    sub = lax.axis_index("s")
    wid = sub * NC + core
    ones16 = jnp.full((16,), 1.0, _f32)
    zero16 = jnp.zeros((16,), _f32)

    @pl.loop(0, C, step=16)
    def _(g):
        onesv[pl.ds(g, 16)] = ones16

    @pl.loop(0, ELEMS_PER_TILE, step=16)
    def _(g):
        zerov[pl.ds(g, 16)] = zero16

    pltpu.sync_copy(zerov, cnt_sh.at[pl.ds(sub * ELEMS_PER_TILE, ELEMS_PER_TILE)])
    plsc.subcore_barrier()

    for i0 in (0, 1):
        pltpu.async_copy(ckey_hbm.at[pl.ds((wid + i0 * NW) * C, C)],
                         KV[i0], SI[i0])

    @pl.loop(0, PAIRS)
    def _(p):
        for off in (0, 1):
            b = off
            i = 2 * p + off
            cid = wid + i * NW
            cid2 = cid + 2 * NW

            @pl.when(cid < CHUNKS)
            def _():
                pltpu.make_async_copy(ckey_hbm.at[pl.ds(cid * C, C)],
                                      KV[b], SI[b]).wait()
                pltpu.async_copy(onesv, cnt_sh.at[KV[b]], SO[b], add=True)

            @pl.when(cid2 < CHUNKS)
            def _():
                pltpu.make_async_copy(onesv, cnt_sh.at[KV[b]], SO[b]).wait()
                pltpu.async_copy(ckey_hbm.at[pl.ds(cid2 * C, C)], KV[b], SI[b])

    pltpu.make_async_copy(onesv, cnt_sh.at[KV[0]], SO[0]).wait()
    pltpu.make_async_copy(onesv, cnt_sh.at[KV[1]], SO[1]).wait()

    plsc.subcore_barrier()
    pltpu.sync_copy(cnt_sh.at[pl.ds(sub * ELEMS_PER_TILE, ELEMS_PER_TILE)], zerov)
    pltpu.sync_copy(zerov,
                    out_hbm.at[pl.ds(core * KEYS_PAD + sub * ELEMS_PER_TILE,
                                     ELEMS_PER_TILE)])


_sc_count = pl.kernel(
    _sc_count_body,
    out_type=jax.ShapeDtypeStruct((NC * KEYS_PAD,), _f32),
    mesh=_mesh,
    scratch_types=[
        pltpu.VMEM((C,), _i32),
        pltpu.VMEM((C,), _i32),
        pltpu.VMEM((C,), _f32),
        pltpu.VMEM((ELEMS_PER_TILE,), _f32),
        pltpu.VMEM_SHARED((KEYS_PAD,), _f32),
        pltpu.SemaphoreType.DMA,
        pltpu.SemaphoreType.DMA,
        pltpu.SemaphoreType.DMA,
        pltpu.SemaphoreType.DMA,
    ],
    compiler_params=_sc_params,
)


def _sc_main_body(pk_hbm, z_hbm, cnt_hbm, out_hbm,
                  kb0, kb1, sd0, sd1,
                  sc0, sc1, sw0, sw1, r0, r1,
                  si0, si1, ss0, ss1, sz0, sz1, so0, so1,
                  acc_sh):
    KB = (kb0, kb1)
    SD = (sd0, sd1)
    SCV = (sc0, sc1)
    SCW = (sw0, sw1)
    RW = (r0, r1)
    SI = (si0, si1)
    SS = (ss0, ss1)
    SZ = (sz0, sz1)
    SO = (so0, so1)
    core = lax.axis_index("c")
    sub = lax.axis_index("s")
    wid = sub * NC + core
    zero_row = jnp.zeros((16,), _f32)

    @pl.loop(0, C)
    def _(j):
        for k in range(8):
            r0[j, pl.ds(k * 16, 16)] = zero_row

    for k in range(4):
        pltpu.sync_copy(r0, acc_sh.at[pl.ds(sub * NPT + k * 128, 128)])
    pltpu.sync_copy(r0.at[pl.ds(0, 112)],
                    acc_sh.at[pl.ds(sub * NPT + 512, 112)])

    @pl.when(sub == NS - 1)
    def _():
        pltpu.sync_copy(r0.at[pl.ds(0, 16)], acc_sh.at[pl.ds(NS * NPT, 16)])

    plsc.subcore_barrier()

    def idx_issue(i, b):
        cid = wid + i * NW

        @pl.when(cid < CHUNKS)
        def _():
            pltpu.async_copy(pk_hbm.at[cid], KB[b], SI[b])

    def gathers_issue(i, b):
        cid = wid + i * NW

        @pl.when(cid < CHUNKS)
        def _():
            pltpu.make_async_copy(pk_hbm.at[cid], KB[b], SI[b]).wait()

            @pl.when(i >= 2)
            def _():
                pltpu.make_async_copy(RW[b], acc_sh.at[SD[b]], SO[b]).wait()

            pltpu.async_copy(cnt_hbm.at[KB[b].at[pl.ds(C, C)]], SCV[b], SS[b])
            pltpu.async_copy(cnt_hbm.at[KB[b].at[pl.ds(2 * C, C)]], SCW[b], SS[b])
            pltpu.async_copy(z_hbm.at[KB[b].at[pl.ds(0, C)]], RW[b], SZ[b])

    def process(i, b):
        cid = wid + i * NW

        @pl.when(cid < CHUNKS)
        def _():
            pltpu.make_async_copy(cnt_hbm.at[KB[b].at[pl.ds(C, C)]],
                                  SCV[b], SS[b]).wait()
            pltpu.make_async_copy(cnt_hbm.at[KB[b].at[pl.ds(2 * C, C)]],
                                  SCW[b], SS[b]).wait()
            pltpu.make_async_copy(z_hbm.at[KB[b].at[pl.ds(0, C)]],
                                  RW[b], SZ[b]).wait()

            @pl.loop(0, C, step=16)
            def _(g):
                cval = SCV[b][pl.ds(g, 16)] + SCW[b][pl.ds(g, 16)]
                SCV[b][pl.ds(g, 16)] = 1.0 / jnp.maximum(cval, 1.0)

            @pl.loop(0, C, step=16)
            def _(g):
                SD[b][pl.ds(g, 16)] = KB[b][pl.ds(3 * C + g, 16)]

        idx_issue(i + 2, b)

        @pl.when(cid < CHUNKS)
        def _():
            @pl.loop(0, C, step=4)
            def _(j):
                s16 = [plsc.load_gather(SCV[b], [lax.broadcast(j + u, (16,))])
                       for u in range(4)]
                for u in range(4):
                    for k in range(8):
                        RW[b][j + u, pl.ds(k * 16, 16)] = (
                            RW[b][j + u, pl.ds(k * 16, 16)] * s16[u])

            pltpu.async_copy(RW[b], acc_sh.at[SD[b]], SO[b], add=True)

    idx_issue(0, 0)
    idx_issue(1, 1)
    gathers_issue(0, 0)

    @pl.loop(0, PAIRS)
    def _(p):
        for off in (0, 1):
            i = 2 * p + off
            gathers_issue(i + 1, (off + 1) % 2)
            process(i, off)

    pltpu.make_async_copy(RW[0], acc_sh.at[SD[0]], SO[0]).wait()
    pltpu.make_async_copy(RW[1], acc_sh.at[SD[1]], SO[1]).wait()

    plsc.subcore_barrier()
    for k in range(4):
        pltpu.sync_copy(acc_sh.at[pl.ds(sub * NPT + k * 128, 128)], r0)
        pltpu.sync_copy(r0, out_hbm.at[core, pl.ds(sub * NPT + k * 128, 128)])
    pltpu.sync_copy(acc_sh.at[pl.ds(sub * NPT + 512, 112)], r0.at[pl.ds(0, 112)])
    pltpu.sync_copy(r0.at[pl.ds(0, 112)],
                    out_hbm.at[core, pl.ds(sub * NPT + 512, 112)])

    @pl.when(sub == NS - 1)
    def _():
        pltpu.sync_copy(acc_sh.at[pl.ds(NS * NPT, 16)], r0.at[pl.ds(0, 16)])
        pltpu.sync_copy(r0.at[pl.ds(0, 16)],
                        out_hbm.at[core, pl.ds(NS * NPT, 16)])


_sc_main = pl.kernel(
    _sc_main_body,
    out_type=jax.ShapeDtypeStruct((NC, N, D), _f32),
    mesh=_mesh,
    scratch_types=[
        pltpu.VMEM((4 * C,), _i32),
        pltpu.VMEM((4 * C,), _i32),
        pltpu.VMEM((C,), _i32),
        pltpu.VMEM((C,), _i32),
        pltpu.VMEM((C,), _f32),
        pltpu.VMEM((C,), _f32),
        pltpu.VMEM((C,), _f32),
        pltpu.VMEM((C,), _f32),
        pltpu.VMEM((C, D), _f32),
        pltpu.VMEM((C, D), _f32),
        pltpu.SemaphoreType.DMA,
        pltpu.SemaphoreType.DMA,
        pltpu.SemaphoreType.DMA,
        pltpu.SemaphoreType.DMA,
        pltpu.SemaphoreType.DMA,
        pltpu.SemaphoreType.DMA,
        pltpu.SemaphoreType.DMA,
        pltpu.SemaphoreType.DMA,
        pltpu.VMEM_SHARED((N, D), _f32),
    ],
    compiler_params=_sc_params,
)


def _tc_final_body(part_ref, root_ref, out_ref):
    p = part_ref[...]
    out_ref[...] = p[0] + p[1] + root_ref[...]


def _tc_final(parts, root):
    nb = 10
    bn = N // nb
    return pl.pallas_call(
        _tc_final_body,
        grid=(nb,),
        in_specs=[
            pl.BlockSpec((NC, bn, D), lambda i: (0, i, 0)),
            pl.BlockSpec((bn, D), lambda i: (i, 0)),
        ],
        out_specs=pl.BlockSpec((bn, D), lambda i: (i, 0)),
        out_shape=jax.ShapeDtypeStruct((N, D), _f32),
    )(parts, root)


def kernel(x_src, x_target, edge_index, edge_type, target_node_type,
           src_node_type, W_rel, W_root, b_root):
    edge_index = edge_index.astype(_i32)
    edge_type = edge_type.astype(_i32)
    tnt2d = target_node_type.astype(_i32).reshape(N, 1)
    Wcat = jnp.transpose(W_rel.astype(_f32), (2, 0, 1)).reshape(D, R * D)
    Wrcat = jnp.transpose(W_root.astype(_f32), (2, 0, 1)).reshape(D, T * D)

    packed, ckey = _tc_keys(edge_index, edge_type)
    cnt = _sc_count(ckey)

    z, root = _tc_precompute(x_src.astype(_f32), x_target.astype(_f32),
                             Wcat, Wrcat, b_root.astype(_f32), tnt2d)
    z = z.reshape(KEYS, D)

    parts = _sc_main(packed, z, cnt)
    return _tc_final(parts, root)
